# baseline (speedup 1.0000x reference)
import functools

import jax
import jax.numpy as jnp
from jax import lax
from jax.experimental import pallas as pl
from jax.experimental.pallas import tpu as pltpu
from jax.experimental.pallas import tpu_sc as plsc

N_NODES = 10000
N_EDGES = 160000
D_IN = 256
D_OUT = 256

NUM_CORES = 2
NUM_SUBCORES = 16
NUM_TILES = NUM_CORES * NUM_SUBCORES
LANES = 16

EPT = N_EDGES // NUM_TILES
EPT_BUF = ((EPT + LANES - 1) // LANES) * LANES
EC = 128
EPT_PAD = ((EPT + EC - 1) // EC) * EC
N_PAD = 10240
NPASS = 10
ROWS_PASS = N_PAD // NPASS
RPT = ROWS_PASS // NUM_SUBCORES
WORDS = D_OUT // 2


def _matmul_body(x_ref, w_ref, o_ref):
    o_ref[...] = jnp.dot(x_ref[...], w_ref[...],
                         preferred_element_type=jnp.float32
                         ).astype(jnp.bfloat16)


def _matmul_bf16(x, W2):
    BR = 2000
    return pl.pallas_call(
        _matmul_body,
        grid=(N_NODES // BR,),
        in_specs=[
            pl.BlockSpec((BR, D_IN), lambda i: (i, 0)),
            pl.BlockSpec((D_IN, D_OUT), lambda i: (0, 0)),
        ],
        out_specs=pl.BlockSpec((BR, D_OUT), lambda i: (i, 0)),
        out_shape=jax.ShapeDtypeStruct((N_NODES, D_OUT), jnp.bfloat16),
    )(x, W2)


def _sumrelu_body(a0_ref, a1_ref, b0_ref, b1_ref, o_ref):
    h = D_OUT // 2
    o_ref[:, :h] = jnp.maximum(a0_ref[0, 0] + b0_ref[0, 0], 0.0)
    o_ref[:, h:] = jnp.maximum(a1_ref[0, 0] + b1_ref[0, 0], 0.0)


def _sum_relu(parts):
    BR = 1024
    spec = lambda c, h: pl.BlockSpec((1, 1, BR, D_OUT // 2),
                                     lambda i, c=c, h=h: (c, h, i, 0))
    return pl.pallas_call(
        _sumrelu_body,
        grid=(N_PAD // BR,),
        in_specs=[spec(0, 0), spec(0, 1), spec(1, 0), spec(1, 1)],
        out_specs=pl.BlockSpec((BR, D_OUT), lambda i: (i, 0)),
        out_shape=jax.ShapeDtypeStruct((N_PAD, D_OUT), jnp.float32),
    )(parts, parts, parts, parts)


def _sc_body(pre_hbm, col_hbm, row_hbm, val_hbm, out_hbm,
             colt, rowt, valt, colc, rowc, valc,
             cbuf, rbuf, mbuf, cbuf2, rbuf2, mbuf2, msgsA, msgsB,
             accA, accB, gsem, ssem):
    cid = lax.axis_index("c")
    sid = lax.axis_index("s")
    ebase = (cid * NUM_SUBCORES + sid) * EPT
    ngrp = EPT_BUF // LANES
    zeros16i = jnp.zeros((LANES,), jnp.int32)
    zeros16f = jnp.zeros((LANES,), jnp.float32)
    himask = jnp.int32(-65536)

    pltpu.sync_copy(col_hbm.at[pl.ds(ebase, EPT)], colt.at[pl.ds(0, EPT)])
    pltpu.sync_copy(row_hbm.at[pl.ds(ebase, EPT)], rowt.at[pl.ds(0, EPT)])
    pltpu.sync_copy(val_hbm.at[pl.ds(ebase, EPT)], valt.at[pl.ds(0, EPT)])
    tail = rowt[pl.ds(EPT - 8, LANES)]
    lane = lax.iota(jnp.int32, LANES)
    rowt[pl.ds(EPT - 8, LANES)] = jnp.where(lane < 8, tail, -1)

    def prefill(g, _):
        colc[pl.ds(g * LANES, LANES)] = zeros16i
        rowc[pl.ds(g * LANES, LANES)] = zeros16i
        return 0
    lax.fori_loop(0, EPT_PAD // LANES, prefill, 0)

    def run_pass(p, _):
        lo = p * ROWS_PASS

        def zfill(r, _):
            for j in range(D_OUT // (2 * LANES)):
                msgsA[r, pl.ds(j * LANES, LANES)] = zeros16f
            return 0
        lax.fori_loop(0, RPT, zfill, 0)
        pltpu.sync_copy(msgsA.at[pl.ds(0, RPT), :],
                        accA.at[pl.ds(sid * RPT, RPT), :])
        pltpu.sync_copy(msgsA.at[pl.ds(0, RPT), :],
                        accB.at[pl.ds(sid * RPT, RPT), :])

        def vfill(g, _):
            valc[pl.ds(g * LANES, LANES)] = zeros16f
            return 0
        lax.fori_loop(0, EPT_PAD // LANES, vfill, 0)

        def compact(g, cnt):
            sl = pl.ds(g * LANES, LANES)
            rows = rowt[sl]
            mask = (rows >= lo) & (rows < lo + ROWS_PASS)
            plsc.store_compressed(colc.at[pl.ds(cnt, LANES)],
                                  colt[sl], mask=mask)
            plsc.store_compressed(rowc.at[pl.ds(cnt, LANES)],
                                  rows - lo, mask=mask)
            plsc.store_compressed(valc.at[pl.ds(cnt, LANES)],
                                  valt[sl], mask=mask)
            return cnt + plsc.all_reduce_population_count(mask)[0]
        cnt = lax.fori_loop(0, ngrp, compact, 0)
        plsc.subcore_barrier()

        nchunk = (cnt + EC - 1) // EC

        def fill_and_gather(c, cb, rb, mb):
            base = c * EC
            for k in range(EC // LANES):
                sl = pl.ds(base + k * LANES, LANES)
                dl = pl.ds(k * LANES, LANES)
                cb[dl] = colc[sl]
                rb[dl] = rowc[sl]
            pltpu.async_copy(pre_hbm.at[cb], mb, gsem)

        def scale_msgs(c, mb):
            base = c * EC

            def scale(g, _):
                vvals = valc[pl.ds(base + g * LANES, LANES)]
                for l in range(LANES):
                    v = vvals[l]
                    e = g * LANES + l
                    for j in range(WORDS // LANES):
                        w = mb[e, pl.ds(j * LANES, LANES)]
                        flo = plsc.bitcast(w << 16, jnp.float32)
                        fhi = plsc.bitcast(w & himask, jnp.float32)
                        tgt = msgsA if j < WORDS // (2 * LANES) else msgsB
                        o = (2 * j * LANES) % (D_OUT // 2)
                        tgt[e, pl.ds(o, LANES)] = flo * v
                        tgt[e, pl.ds(o + LANES, LANES)] = fhi * v
                return 0
            lax.fori_loop(0, EC // LANES, scale, 0)

        @pl.when(nchunk > 0)
        def _():
            fill_and_gather(0, cbuf, rbuf, mbuf)

        def chunk(c, _):
            def stage(mb, rb, cb2, rb2, mb2):
                @pl.when(c >= 1)
                def _():
                    pltpu.make_async_copy(msgsA, accA.at[rbuf], ssem).wait()
                    pltpu.make_async_copy(msgsB, accB.at[rbuf], ssem).wait()

                pltpu.make_async_copy(pre_hbm.at[cbuf], mb, gsem).wait()

                @pl.when(c + 1 < nchunk)
                def _():
                    fill_and_gather(c + 1, cb2, rb2, mb2)

                scale_msgs(c, mb)
                pltpu.async_copy(msgsA, accA.at[rb], ssem, add=True)
                pltpu.async_copy(msgsB, accB.at[rb], ssem, add=True)

            @pl.when(c % 2 == 0)
            def _():
                stage(mbuf, rbuf, cbuf2, rbuf2, mbuf2)

            @pl.when(c % 2 == 1)
            def _():
                stage(mbuf2, rbuf2, cbuf, rbuf, mbuf)
            return 0
        lax.fori_loop(0, nchunk, chunk, 0)

        @pl.when(nchunk > 0)
        def _():
            pltpu.make_async_copy(msgsA, accA.at[rbuf], ssem).wait()
            pltpu.make_async_copy(msgsB, accB.at[rbuf], ssem).wait()
        plsc.subcore_barrier()

        pltpu.sync_copy(
            accA.at[pl.ds(sid * RPT, RPT), :],
            out_hbm.at[cid, 0, pl.ds(lo + sid * RPT, RPT), :])
        pltpu.sync_copy(
            accB.at[pl.ds(sid * RPT, RPT), :],
            out_hbm.at[cid, 1, pl.ds(lo + sid * RPT, RPT), :])
        return 0
    lax.fori_loop(0, NPASS, run_pass, 0)


_sc_spmm = functools.partial(
    pl.kernel,
    mesh=plsc.VectorSubcoreMesh(core_axis_name="c", subcore_axis_name="s"),
    compiler_params=pltpu.CompilerParams(needs_layout_passes=False),
    out_type=jax.ShapeDtypeStruct((NUM_CORES, 2, N_PAD, D_OUT // 2),
                                  jnp.float32),
    scratch_types=[
        pltpu.VMEM((EPT_BUF,), jnp.int32),
        pltpu.VMEM((EPT_BUF,), jnp.int32),
        pltpu.VMEM((EPT_BUF,), jnp.float32),
        pltpu.VMEM((EPT_PAD,), jnp.int32),
        pltpu.VMEM((EPT_PAD,), jnp.int32),
        pltpu.VMEM((EPT_PAD,), jnp.float32),
        pltpu.VMEM((EC,), jnp.int32),
        pltpu.VMEM((EC,), jnp.int32),
        pltpu.VMEM((EC, WORDS), jnp.int32),
        pltpu.VMEM((EC,), jnp.int32),
        pltpu.VMEM((EC,), jnp.int32),
        pltpu.VMEM((EC, WORDS), jnp.int32),
        pltpu.VMEM((EC, D_OUT // 2), jnp.float32),
        pltpu.VMEM((EC, D_OUT // 2), jnp.float32),
        pltpu.VMEM_SHARED((ROWS_PASS, D_OUT // 2), jnp.float32),
        pltpu.VMEM_SHARED((ROWS_PASS, D_OUT // 2), jnp.float32),
        pltpu.SemaphoreType.DMA,
        pltpu.SemaphoreType.DMA,
    ],
)(_sc_body)


def _bf16_pair_perm():
    inv = [0] * D_OUT
    for j in range(D_OUT // 32):
        for k in range(LANES):
            inv[32 * j + 2 * k] = 32 * j + k
            inv[32 * j + 2 * k + 1] = 32 * j + 16 + k
    return inv


_PERM = _bf16_pair_perm()


def kernel(x, edge_index, edge_values, W):
    row = edge_index[0].astype(jnp.int32)
    col = edge_index[1].astype(jnp.int32)
    W2 = W[:, jnp.array(_PERM, dtype=jnp.int32)]
    pre_bf = _matmul_bf16(x, W2)
    pre = lax.bitcast_convert_type(
        pre_bf.reshape(N_NODES, WORDS, 2), jnp.int32)
    parts = _sc_spmm(pre, col, row, edge_values)
    return _sum_relu(parts)[:N_NODES]


# speedup vs baseline: 2.3300x; 2.3300x over previous
import functools

import jax
import jax.numpy as jnp
from jax import lax
from jax.experimental import pallas as pl
from jax.experimental.pallas import tpu as pltpu
from jax.experimental.pallas import tpu_sc as plsc

N_NODES = 10000
N_EDGES = 160000
D_IN = 256
D_OUT = 256
D_HALF = D_OUT // 2

NUM_CORES = 2
NUM_SUBCORES = 16
LANES = 16

EPT = N_EDGES // NUM_SUBCORES
EC = 128
EPT_PAD = ((EPT + EC - 1) // EC) * EC
N_PAD = 10240
NPASS = 4
ROWS_PASS = N_PAD // NPASS
RPT = ROWS_PASS // NUM_SUBCORES
WB = 80


def _matmul_body(x_ref, w_ref, o_ref):
    o_ref[0, :, :] = jnp.dot(x_ref[...], w_ref[...],
                             preferred_element_type=jnp.float32)


def _matmul_halves(x, W):
    BR = 1000
    grid = (N_NODES // BR, NUM_CORES)
    return pl.pallas_call(
        _matmul_body,
        grid=grid,
        in_specs=[
            pl.BlockSpec((BR, D_IN), lambda i, j: (i, 0)),
            pl.BlockSpec((D_IN, D_HALF), lambda i, j: (0, j)),
        ],
        out_specs=pl.BlockSpec((1, BR, D_HALF), lambda i, j: (j, i, 0)),
        out_shape=jax.ShapeDtypeStruct((NUM_CORES, N_NODES, D_HALF),
                                       jnp.float32),
    )(x, W)


def _sc_body(pre_hbm, col_hbm, row_hbm, val_hbm, out_hbm,
             colt, rowt, valt, colc, rowc, valc,
             cbuf, rbuf, msgs, cbuf2, rbuf2, msgs2, acc, gsem, ssem):
    cid = lax.axis_index("c")
    sid = lax.axis_index("s")
    ebase = sid * EPT
    ngrp = EPT // LANES
    zeros16i = jnp.zeros((LANES,), jnp.int32)
    zeros16f = jnp.zeros((LANES,), jnp.float32)

    pltpu.sync_copy(col_hbm.at[pl.ds(ebase, EPT)], colt)
    pltpu.sync_copy(row_hbm.at[pl.ds(ebase, EPT)], rowt)
    pltpu.sync_copy(val_hbm.at[pl.ds(ebase, EPT)], valt)

    def prefill(g, _):
        colc[pl.ds(g * LANES, LANES)] = zeros16i
        rowc[pl.ds(g * LANES, LANES)] = zeros16i
        return 0
    lax.fori_loop(0, EPT_PAD // LANES, prefill, 0)

    coff = cid * N_NODES

    def run_pass(p, _):
        lo = p * ROWS_PASS

        def zfill(r, _):
            for j in range(D_HALF // LANES):
                msgs[r, pl.ds(j * LANES, LANES)] = zeros16f
            return 0
        lax.fori_loop(0, WB, zfill, 0)
        for b in range(RPT // WB):
            pltpu.sync_copy(
                msgs.at[pl.ds(0, WB), :],
                acc.at[pl.ds(sid * RPT + b * WB, WB), :])

        def vfill(g, _):
            valc[pl.ds(g * LANES, LANES)] = zeros16f
            return 0
        lax.fori_loop(0, EPT_PAD // LANES, vfill, 0)

        def compact(g, cnt):
            sl = pl.ds(g * LANES, LANES)
            rows = rowt[sl]
            mask = (rows >= lo) & (rows < lo + ROWS_PASS)
            plsc.store_compressed(colc.at[pl.ds(cnt, LANES)],
                                  colt[sl] + coff, mask=mask)
            plsc.store_compressed(rowc.at[pl.ds(cnt, LANES)],
                                  rows - lo, mask=mask)
            plsc.store_compressed(valc.at[pl.ds(cnt, LANES)],
                                  valt[sl], mask=mask)
            return cnt + plsc.all_reduce_population_count(mask)[0]
        cnt = lax.fori_loop(0, ngrp, compact, 0)
        plsc.subcore_barrier()

        nchunk = (cnt + EC - 1) // EC

        def fill_and_gather(c, cb, rb, ms):
            base = c * EC
            for k in range(EC // LANES):
                sl = pl.ds(base + k * LANES, LANES)
                dl = pl.ds(k * LANES, LANES)
                cb[dl] = colc[sl]
                rb[dl] = rowc[sl]
            pltpu.async_copy(pre_hbm.at[cb], ms, gsem)

        def scale_msgs(c, ms):
            base = c * EC

            def scale(g, _):
                vvals = valc[pl.ds(base + g * LANES, LANES)]
                for l in range(LANES):
                    v = vvals[l]
                    e = g * LANES + l
                    for j in range(D_HALF // LANES):
                        fl = pl.ds(j * LANES, LANES)
                        ms[e, fl] = ms[e, fl] * v
                return 0
            lax.fori_loop(0, EC // LANES, scale, 0)

        @pl.when(nchunk > 0)
        def _():
            fill_and_gather(0, cbuf, rbuf, msgs)

        def chunk(c, _):
            def stage(ms, rb, cb2, rb2, ms2):
                pltpu.make_async_copy(pre_hbm.at[cbuf], ms, gsem).wait()

                @pl.when(c + 1 < nchunk)
                def _():
                    @pl.when(c >= 1)
                    def _():
                        pltpu.make_async_copy(ms2, acc.at[rb2], ssem).wait()
                    fill_and_gather(c + 1, cb2, rb2, ms2)

                scale_msgs(c, ms)
                pltpu.async_copy(ms, acc.at[rb], ssem, add=True)

            @pl.when(c % 2 == 0)
            def _():
                stage(msgs, rbuf, cbuf2, rbuf2, msgs2)

            @pl.when(c % 2 == 1)
            def _():
                stage(msgs2, rbuf2, cbuf, rbuf, msgs)
            return 0
        lax.fori_loop(0, nchunk, chunk, 0)

        @pl.when(nchunk > 0)
        def _():
            pltpu.make_async_copy(msgs, acc.at[rbuf], ssem).wait()

        @pl.when(nchunk > 1)
        def _():
            pltpu.make_async_copy(msgs2, acc.at[rbuf2], ssem).wait()
        plsc.subcore_barrier()

        for b in range(RPT // WB):
            r0 = sid * RPT + b * WB
            pltpu.sync_copy(acc.at[pl.ds(r0, WB), :],
                            msgs.at[pl.ds(0, WB), :])

            def relu_row(r, _):
                for j in range(D_HALF // LANES):
                    fl = pl.ds(j * LANES, LANES)
                    msgs[r, fl] = jnp.maximum(msgs[r, fl], 0.0)
                return 0
            lax.fori_loop(0, WB, relu_row, 0)
            pltpu.sync_copy(
                msgs.at[pl.ds(0, WB), :],
                out_hbm.at[pl.ds(lo + r0, WB),
                           pl.ds(cid * D_HALF, D_HALF)])
        return 0
    lax.fori_loop(0, NPASS, run_pass, 0)


_sc_spmm = functools.partial(
    pl.kernel,
    mesh=plsc.VectorSubcoreMesh(core_axis_name="c", subcore_axis_name="s"),
    compiler_params=pltpu.CompilerParams(needs_layout_passes=False),
    out_type=jax.ShapeDtypeStruct((N_PAD, D_OUT), jnp.float32),
    scratch_types=[
        pltpu.VMEM((EPT,), jnp.int32),
        pltpu.VMEM((EPT,), jnp.int32),
        pltpu.VMEM((EPT,), jnp.float32),
        pltpu.VMEM((EPT_PAD,), jnp.int32),
        pltpu.VMEM((EPT_PAD,), jnp.int32),
        pltpu.VMEM((EPT_PAD,), jnp.float32),
        pltpu.VMEM((EC,), jnp.int32),
        pltpu.VMEM((EC,), jnp.int32),
        pltpu.VMEM((EC, D_HALF), jnp.float32),
        pltpu.VMEM((EC,), jnp.int32),
        pltpu.VMEM((EC,), jnp.int32),
        pltpu.VMEM((EC, D_HALF), jnp.float32),
        pltpu.VMEM_SHARED((ROWS_PASS, D_HALF), jnp.float32),
        pltpu.SemaphoreType.DMA,
        pltpu.SemaphoreType.DMA,
    ],
)(_sc_body)


def kernel(x, edge_index, edge_values, W):
    row = edge_index[0].astype(jnp.int32)
    col = edge_index[1].astype(jnp.int32)
    pre = _matmul_halves(x, W).reshape(NUM_CORES * N_NODES, D_HALF)
    out = _sc_spmm(pre, col, row, edge_values)
    return out[:N_NODES]


# speedup vs baseline: 2.3405x; 1.0045x over previous
import functools

import jax
import jax.numpy as jnp
from jax import lax
from jax.experimental import pallas as pl
from jax.experimental.pallas import tpu as pltpu
from jax.experimental.pallas import tpu_sc as plsc

N_NODES = 10000
N_EDGES = 160000
D_IN = 256
D_OUT = 256
D_HALF = D_OUT // 2

NUM_CORES = 2
NUM_SUBCORES = 16
LANES = 16

EPT = N_EDGES // NUM_SUBCORES
EC = 128
EPT_PAD = ((EPT + EC - 1) // EC) * EC
N_PAD = 10240
NPASS = 4
ROWS_PASS = N_PAD // NPASS
RPT = ROWS_PASS // NUM_SUBCORES
WB = 80


def _matmul_body(x_ref, w_ref, o_ref):
    o_ref[0, :, :] = jnp.dot(x_ref[...], w_ref[...],
                             preferred_element_type=jnp.float32)


def _matmul_halves(x, W):
    BR = 1000
    grid = (N_NODES // BR, NUM_CORES)
    return pl.pallas_call(
        _matmul_body,
        grid=grid,
        in_specs=[
            pl.BlockSpec((BR, D_IN), lambda i, j: (i, 0)),
            pl.BlockSpec((D_IN, D_HALF), lambda i, j: (0, j)),
        ],
        out_specs=pl.BlockSpec((1, BR, D_HALF), lambda i, j: (j, i, 0)),
        out_shape=jax.ShapeDtypeStruct((NUM_CORES, N_NODES, D_HALF),
                                       jnp.float32),
    )(x, W)


def _sc_body(pre_hbm, col_hbm, row_hbm, val_hbm, out_hbm,
             colt, rowt, valt, colc, rowc, valc,
             cbuf, rbuf, msgs, cbuf2, rbuf2, msgs2, acc, gsem, gsem2, ssem):
    cid = lax.axis_index("c")
    sid = lax.axis_index("s")
    ebase = sid * EPT
    ngrp = EPT // LANES
    zeros16i = jnp.zeros((LANES,), jnp.int32)
    zeros16f = jnp.zeros((LANES,), jnp.float32)

    pltpu.sync_copy(col_hbm.at[pl.ds(ebase, EPT)], colt)
    pltpu.sync_copy(row_hbm.at[pl.ds(ebase, EPT)], rowt)
    pltpu.sync_copy(val_hbm.at[pl.ds(ebase, EPT)], valt)

    def prefill(g, _):
        colc[pl.ds(g * LANES, LANES)] = zeros16i
        rowc[pl.ds(g * LANES, LANES)] = zeros16i
        return 0
    lax.fori_loop(0, EPT_PAD // LANES, prefill, 0)

    coff = cid * N_NODES

    def run_pass(p, _):
        lo = p * ROWS_PASS

        def zfill(r, _):
            for j in range(D_HALF // LANES):
                msgs[r, pl.ds(j * LANES, LANES)] = zeros16f
            return 0
        lax.fori_loop(0, WB, zfill, 0)
        for b in range(RPT // WB):
            pltpu.sync_copy(
                msgs.at[pl.ds(0, WB), :],
                acc.at[pl.ds(sid * RPT + b * WB, WB), :])

        def vfill(g, _):
            valc[pl.ds(g * LANES, LANES)] = zeros16f
            return 0
        lax.fori_loop(0, EPT_PAD // LANES, vfill, 0)

        def compact(g, cnt):
            sl = pl.ds(g * LANES, LANES)
            rows = rowt[sl]
            mask = (rows >= lo) & (rows < lo + ROWS_PASS)
            plsc.store_compressed(colc.at[pl.ds(cnt, LANES)],
                                  colt[sl] + coff, mask=mask)
            plsc.store_compressed(rowc.at[pl.ds(cnt, LANES)],
                                  rows - lo, mask=mask)
            plsc.store_compressed(valc.at[pl.ds(cnt, LANES)],
                                  valt[sl], mask=mask)
            return cnt + plsc.all_reduce_population_count(mask)[0]
        cnt = lax.fori_loop(0, ngrp, compact, 0)
        plsc.subcore_barrier()

        nchunk = (cnt + EC - 1) // EC

        def fill_and_gather(c, cb, rb, ms, sem):
            base = c * EC
            for k in range(EC // LANES):
                sl = pl.ds(base + k * LANES, LANES)
                dl = pl.ds(k * LANES, LANES)
                cb[dl] = colc[sl]
                rb[dl] = rowc[sl]
            pltpu.async_copy(pre_hbm.at[cb], ms, sem)

        def scale_msgs(c, ms):
            base = c * EC

            def scale(g, _):
                vvals = valc[pl.ds(base + g * LANES, LANES)]
                for l in range(LANES):
                    v = vvals[l]
                    e = g * LANES + l
                    for j in range(D_HALF // LANES):
                        fl = pl.ds(j * LANES, LANES)
                        ms[e, fl] = ms[e, fl] * v
                return 0
            lax.fori_loop(0, EC // LANES, scale, 0)

        @pl.when(nchunk > 0)
        def _():
            fill_and_gather(0, cbuf, rbuf, msgs, gsem)

        def chunk(c, _):
            def stage(ms, rb, cb2, rb2, ms2, mysem, othersem):
                @pl.when(c + 1 < nchunk)
                def _():
                    @pl.when(c >= 1)
                    def _():
                        pltpu.make_async_copy(ms2, acc.at[rb2], ssem).wait()
                    fill_and_gather(c + 1, cb2, rb2, ms2, othersem)

                pltpu.make_async_copy(pre_hbm.at[cbuf], ms, mysem).wait()
                scale_msgs(c, ms)
                pltpu.async_copy(ms, acc.at[rb], ssem, add=True)

            @pl.when(c % 2 == 0)
            def _():
                stage(msgs, rbuf, cbuf2, rbuf2, msgs2, gsem, gsem2)

            @pl.when(c % 2 == 1)
            def _():
                stage(msgs2, rbuf2, cbuf, rbuf, msgs, gsem2, gsem)
            return 0
        lax.fori_loop(0, nchunk, chunk, 0)

        @pl.when(nchunk > 0)
        def _():
            pltpu.make_async_copy(msgs, acc.at[rbuf], ssem).wait()

        @pl.when(nchunk > 1)
        def _():
            pltpu.make_async_copy(msgs2, acc.at[rbuf2], ssem).wait()
        plsc.subcore_barrier()

        for b in range(RPT // WB):
            r0 = sid * RPT + b * WB
            pltpu.sync_copy(acc.at[pl.ds(r0, WB), :],
                            msgs.at[pl.ds(0, WB), :])

            def relu_row(r, _):
                for j in range(D_HALF // LANES):
                    fl = pl.ds(j * LANES, LANES)
                    msgs[r, fl] = jnp.maximum(msgs[r, fl], 0.0)
                return 0
            lax.fori_loop(0, WB, relu_row, 0)
            pltpu.sync_copy(
                msgs.at[pl.ds(0, WB), :],
                out_hbm.at[pl.ds(lo + r0, WB),
                           pl.ds(cid * D_HALF, D_HALF)])
        return 0
    lax.fori_loop(0, NPASS, run_pass, 0)


_sc_spmm = functools.partial(
    pl.kernel,
    mesh=plsc.VectorSubcoreMesh(core_axis_name="c", subcore_axis_name="s"),
    compiler_params=pltpu.CompilerParams(needs_layout_passes=False),
    out_type=jax.ShapeDtypeStruct((N_PAD, D_OUT), jnp.float32),
    scratch_types=[
        pltpu.VMEM((EPT,), jnp.int32),
        pltpu.VMEM((EPT,), jnp.int32),
        pltpu.VMEM((EPT,), jnp.float32),
        pltpu.VMEM((EPT_PAD,), jnp.int32),
        pltpu.VMEM((EPT_PAD,), jnp.int32),
        pltpu.VMEM((EPT_PAD,), jnp.float32),
        pltpu.VMEM((EC,), jnp.int32),
        pltpu.VMEM((EC,), jnp.int32),
        pltpu.VMEM((EC, D_HALF), jnp.float32),
        pltpu.VMEM((EC,), jnp.int32),
        pltpu.VMEM((EC,), jnp.int32),
        pltpu.VMEM((EC, D_HALF), jnp.float32),
        pltpu.VMEM_SHARED((ROWS_PASS, D_HALF), jnp.float32),
        pltpu.SemaphoreType.DMA,
        pltpu.SemaphoreType.DMA,
        pltpu.SemaphoreType.DMA,
    ],
)(_sc_body)


def kernel(x, edge_index, edge_values, W):
    row = edge_index[0].astype(jnp.int32)
    col = edge_index[1].astype(jnp.int32)
    pre = _matmul_halves(x, W).reshape(NUM_CORES * N_NODES, D_HALF)
    out = _sc_spmm(pre, col, row, edge_values)
    return out[:N_NODES]


# speedup vs baseline: 2.3770x; 1.0156x over previous
import functools

import jax
import jax.numpy as jnp
from jax import lax
from jax.experimental import pallas as pl
from jax.experimental.pallas import tpu as pltpu
from jax.experimental.pallas import tpu_sc as plsc

N_NODES = 10000
N_EDGES = 160000
D_IN = 256
D_OUT = 256
D_HALF = D_OUT // 2

NUM_CORES = 2
NUM_SUBCORES = 16
LANES = 16

EPT = N_EDGES // NUM_SUBCORES
EC = 128
EPT_PAD = ((EPT + EC - 1) // EC) * EC
N_PAD = 10240
NPASS = 4
ROWS_PASS = N_PAD // NPASS
RPT = ROWS_PASS // NUM_SUBCORES
WB = 80


def _matmul_body(x_ref, w_ref, o_ref):
    o_ref[0, :, :] = jnp.dot(x_ref[...], w_ref[...],
                             preferred_element_type=jnp.float32)


def _matmul_halves(x, W):
    BR = 2000
    grid = (N_NODES // BR, NUM_CORES)
    return pl.pallas_call(
        _matmul_body,
        grid=grid,
        in_specs=[
            pl.BlockSpec((BR, D_IN), lambda i, j: (i, 0)),
            pl.BlockSpec((D_IN, D_HALF), lambda i, j: (0, j)),
        ],
        out_specs=pl.BlockSpec((1, BR, D_HALF), lambda i, j: (j, i, 0)),
        out_shape=jax.ShapeDtypeStruct((NUM_CORES, N_NODES, D_HALF),
                                       jnp.float32),
    )(x, W)


def _sc_body(pre_hbm, col_hbm, row_hbm, val_hbm, out_hbm,
             colt, rowt, valt, colc, rowc, valc,
             cbuf, rbuf, msgs, cbuf2, rbuf2, msgs2, acc, gsem, gsem2, ssem):
    cid = lax.axis_index("c")
    sid = lax.axis_index("s")
    ebase = sid * EPT
    ngrp = EPT // LANES
    zeros16i = jnp.zeros((LANES,), jnp.int32)
    zeros16f = jnp.zeros((LANES,), jnp.float32)

    pltpu.sync_copy(col_hbm.at[pl.ds(ebase, EPT)], colt)
    pltpu.sync_copy(row_hbm.at[pl.ds(ebase, EPT)], rowt)
    pltpu.sync_copy(val_hbm.at[pl.ds(ebase, EPT)], valt)

    def prefill(g, _):
        colc[pl.ds(g * LANES, LANES)] = zeros16i
        rowc[pl.ds(g * LANES, LANES)] = zeros16i
        return 0
    lax.fori_loop(0, EPT_PAD // LANES, prefill, 0)

    coff = cid * N_NODES

    def run_pass(p, _):
        lo = p * ROWS_PASS

        def zfill(r, _):
            for j in range(D_HALF // LANES):
                msgs[r, pl.ds(j * LANES, LANES)] = zeros16f
            return 0
        lax.fori_loop(0, WB, zfill, 0)
        for b in range(RPT // WB):
            pltpu.sync_copy(
                msgs.at[pl.ds(0, WB), :],
                acc.at[pl.ds(sid * RPT + b * WB, WB), :])

        def vfill(g, _):
            valc[pl.ds(g * LANES, LANES)] = zeros16f
            return 0
        lax.fori_loop(0, EPT_PAD // LANES, vfill, 0)

        def compact(g, cnt):
            sl = pl.ds(g * LANES, LANES)
            rows = rowt[sl]
            mask = (rows >= lo) & (rows < lo + ROWS_PASS)
            plsc.store_compressed(colc.at[pl.ds(cnt, LANES)],
                                  colt[sl] + coff, mask=mask)
            plsc.store_compressed(rowc.at[pl.ds(cnt, LANES)],
                                  rows - lo, mask=mask)
            plsc.store_compressed(valc.at[pl.ds(cnt, LANES)],
                                  valt[sl], mask=mask)
            return cnt + plsc.all_reduce_population_count(mask)[0]
        cnt = lax.fori_loop(0, ngrp, compact, 0)
        plsc.subcore_barrier()

        nchunk = (cnt + EC - 1) // EC

        def fill_and_gather(c, cb, rb, ms, sem):
            base = c * EC
            for k in range(EC // LANES):
                sl = pl.ds(base + k * LANES, LANES)
                dl = pl.ds(k * LANES, LANES)
                cb[dl] = colc[sl]
                rb[dl] = rowc[sl]
            pltpu.async_copy(pre_hbm.at[cb], ms, sem)

        def scale_msgs(c, ms):
            base = c * EC

            def scale(g, _):
                vvals = valc[pl.ds(base + g * LANES, LANES)]
                for l in range(LANES):
                    v = vvals[l]
                    e = g * LANES + l
                    for j in range(D_HALF // LANES):
                        fl = pl.ds(j * LANES, LANES)
                        ms[e, fl] = ms[e, fl] * v
                return 0
            lax.fori_loop(0, EC // LANES, scale, 0)

        @pl.when(nchunk > 0)
        def _():
            fill_and_gather(0, cbuf, rbuf, msgs, gsem)

        def chunk(c, _):
            def stage(ms, rb, cb2, rb2, ms2, mysem, othersem):
                @pl.when(c + 1 < nchunk)
                def _():
                    @pl.when(c >= 1)
                    def _():
                        pltpu.make_async_copy(ms2, acc.at[rb2], ssem).wait()
                    fill_and_gather(c + 1, cb2, rb2, ms2, othersem)

                pltpu.make_async_copy(pre_hbm.at[cbuf], ms, mysem).wait()
                scale_msgs(c, ms)
                pltpu.async_copy(ms, acc.at[rb], ssem, add=True)

            @pl.when(c % 2 == 0)
            def _():
                stage(msgs, rbuf, cbuf2, rbuf2, msgs2, gsem, gsem2)

            @pl.when(c % 2 == 1)
            def _():
                stage(msgs2, rbuf2, cbuf, rbuf, msgs, gsem2, gsem)
            return 0
        lax.fori_loop(0, nchunk, chunk, 0)

        @pl.when(nchunk > 0)
        def _():
            pltpu.make_async_copy(msgs, acc.at[rbuf], ssem).wait()

        @pl.when(nchunk > 1)
        def _():
            pltpu.make_async_copy(msgs2, acc.at[rbuf2], ssem).wait()
        plsc.subcore_barrier()

        for b in range(RPT // WB):
            r0 = sid * RPT + b * WB
            pltpu.sync_copy(acc.at[pl.ds(r0, WB), :],
                            msgs.at[pl.ds(0, WB), :])

            def relu_row(r, _):
                for j in range(D_HALF // LANES):
                    fl = pl.ds(j * LANES, LANES)
                    msgs[r, fl] = jnp.maximum(msgs[r, fl], 0.0)
                return 0
            lax.fori_loop(0, WB, relu_row, 0)
            pltpu.sync_copy(
                msgs.at[pl.ds(0, WB), :],
                out_hbm.at[pl.ds(lo + r0, WB),
                           pl.ds(cid * D_HALF, D_HALF)])
        return 0
    lax.fori_loop(0, NPASS, run_pass, 0)


_sc_spmm = functools.partial(
    pl.kernel,
    mesh=plsc.VectorSubcoreMesh(core_axis_name="c", subcore_axis_name="s"),
    compiler_params=pltpu.CompilerParams(needs_layout_passes=False),
    out_type=jax.ShapeDtypeStruct((N_PAD, D_OUT), jnp.float32),
    scratch_types=[
        pltpu.VMEM((EPT,), jnp.int32),
        pltpu.VMEM((EPT,), jnp.int32),
        pltpu.VMEM((EPT,), jnp.float32),
        pltpu.VMEM((EPT_PAD,), jnp.int32),
        pltpu.VMEM((EPT_PAD,), jnp.int32),
        pltpu.VMEM((EPT_PAD,), jnp.float32),
        pltpu.VMEM((EC,), jnp.int32),
        pltpu.VMEM((EC,), jnp.int32),
        pltpu.VMEM((EC, D_HALF), jnp.float32),
        pltpu.VMEM((EC,), jnp.int32),
        pltpu.VMEM((EC,), jnp.int32),
        pltpu.VMEM((EC, D_HALF), jnp.float32),
        pltpu.VMEM_SHARED((ROWS_PASS, D_HALF), jnp.float32),
        pltpu.SemaphoreType.DMA,
        pltpu.SemaphoreType.DMA,
        pltpu.SemaphoreType.DMA,
    ],
)(_sc_body)


def kernel(x, edge_index, edge_values, W):
    row = edge_index[0].astype(jnp.int32)
    col = edge_index[1].astype(jnp.int32)
    pre = _matmul_halves(x, W).reshape(NUM_CORES * N_NODES, D_HALF)
    out = _sc_spmm(pre, col, row, edge_values)
    return out[:N_NODES]
